# baseline (speedup 1.0000x reference)
import jax
import jax.numpy as jnp
import numpy as np
from jax.experimental import pallas as pl
from jax.experimental.pallas import tpu as pltpu

_SIZE = 512
_NPOS = 100000
_NROW = 1024
_NLEVELS = 17
_CROWS = 520
_CCOLS = 640

_U = np.uint32(0x3EFA3824).view(np.float32)
_OMU = np.float32(np.float32(1.0) - _U)


def _body(t2_ref, ind_ref, img_ref, out_ref, innert_ref, crop_v,
          ind_v, sem_ind, sem0, sem1):
    acc = jnp.zeros((8, 128), jnp.float32)
    for j in range(128):
        acc = acc + t2_ref[j]
        innert_ref[j] = acc

    lane = jax.lax.broadcasted_iota(jnp.int32, (8, 128), 1)
    v = acc
    for j in range(1, 128):
        rolled = pltpu.roll(v, 1, 1)
        v = jnp.where(lane == j, v + rolled, v)
    inner2 = v

    si = jax.lax.broadcasted_iota(jnp.int32, (8, 128), 0)
    li = jax.lax.broadcasted_iota(jnp.int32, (8, 128), 1)

    def _extract(arr, a, b):
        return jnp.sum(jnp.where((si == a) & (li == b), arr,
                                 jnp.zeros_like(arr)))

    bb = inner2
    for k in range(1, 7):
        rolled = pltpu.roll(bb, 1, 0)
        bb = jnp.where(si == k, bb + rolled, bb)
    o2e_v = pltpu.roll(bb, 1, 0)
    o2e_v = jnp.where(si == 0, jnp.float32(0.0), o2e_v)
    o2e_bc = jnp.broadcast_to(o2e_v[:, 127:128], (8, 128))

    oiv = o2e_bc + inner2
    r1 = pltpu.roll(oiv, 1, 1)
    r2 = pltpu.roll(r1, 1, 0)
    oev = jnp.where(li == 0, r2, r1)
    oev = jnp.where((li == 0) & (si == 0), jnp.float32(0.0), oev)

    s_total = _extract(oev + innert_ref[95], 6, 13)
    r_thr = s_total * _OMU

    r_bc = jnp.full((8, 128), r_thr)
    rlin = si * 128 + li
    m_le95 = rlin <= 781
    m_ge96 = rlin <= 780
    big = jnp.full((8, 128), jnp.int32(1 << 30))
    nacc = 4
    cnts = [jnp.zeros((8, 128), jnp.int32) for _ in range(nacc)]
    fges = [big for _ in range(nacc)]
    for j in range(128):
        mj = m_le95 if j <= 95 else m_ge96
        cv = oev + innert_ref[j]
        lt = (cv < r_bc) & mj
        ge = (cv >= r_bc) & mj
        k = j % nacc
        cnts[k] = cnts[k] + jnp.where(lt, 1, 0)
        fges[k] = jnp.minimum(fges[k], jnp.where(ge, rlin * 128 + j, big))
    cnt_all = cnts[0] + cnts[1] + cnts[2] + cnts[3]
    fge_all = jnp.minimum(jnp.minimum(fges[0], fges[1]),
                          jnp.minimum(fges[2], fges[3]))
    a_cnt = jnp.sum(cnt_all)
    first_ge = jnp.min(fge_all)

    def _slow_search():
        def cumval(mid):
            r = mid // 128
            j = mid % 128
            return _extract(oev + innert_ref[j], r // 128, r % 128)
        low = jnp.int32(0)
        high = jnp.int32(_NPOS)
        for _ in range(_NLEVELS):
            mid = low + (high - low) // 2
            go_left = r_thr <= cumval(mid)
            low = jnp.where(go_left, low, mid)
            high = jnp.where(go_left, mid, high)
        return high

    pos = jax.lax.cond(first_ge == a_cnt, lambda: a_cnt, _slow_search)

    pos0 = pl.multiple_of((pos // 8) * 8, 8)
    cp = pltpu.make_async_copy(ind_ref.at[pl.ds(pos0, 8), :], ind_v, sem_ind)
    cp.start()
    cp.wait()
    iv = ind_v[...]
    si2 = jax.lax.broadcasted_iota(jnp.int32, (8, 2), 0)
    li2 = jax.lax.broadcasted_iota(jnp.int32, (8, 2), 1)
    zero = jnp.zeros_like(iv)
    y = jnp.sum(jnp.where((si2 == pos - pos0) & (li2 == 0), iv, zero))
    x = jnp.sum(jnp.where((si2 == pos - pos0) & (li2 == 1), iv, zero))

    y0 = pl.multiple_of((y // 8) * 8, 8)
    x0 = pl.multiple_of((x // 128) * 128, 128)
    dy = y - y0
    dx = x - x0
    c0 = pltpu.make_async_copy(
        img_ref.at[0, pl.ds(y0, _CROWS), pl.ds(x0, _CCOLS)],
        crop_v.at[0], sem0)
    c1 = pltpu.make_async_copy(
        img_ref.at[1, pl.ds(y0, _CROWS), pl.ds(x0, _CCOLS)],
        crop_v.at[1], sem1)
    c0.start()
    c1.start()
    c0.wait()
    c1.wait()
    for c in range(2):
        arr = crop_v[c]
        arr = pltpu.roll(arr, (_CROWS - dy) % _CROWS, 0)
        arr = pltpu.roll(arr, (_CCOLS - dx) % _CCOLS, 1)
        out_ref[c] = arr[:_SIZE, :_SIZE]


def kernel(img, pmap, ind):
    xp = jnp.pad(pmap, (0, _NROW * 128 - _NPOS))
    t2 = xp.reshape(_NROW, 128).T.reshape(128, 8, 128)

    return pl.pallas_call(
        _body,
        in_specs=[
            pl.BlockSpec(memory_space=pltpu.VMEM),
            pl.BlockSpec(memory_space=pl.ANY),
            pl.BlockSpec(memory_space=pl.ANY),
        ],
        out_specs=pl.BlockSpec(memory_space=pltpu.VMEM),
        out_shape=jax.ShapeDtypeStruct((2, _SIZE, _SIZE), jnp.float32),
        scratch_shapes=[
            pltpu.VMEM((128, 8, 128), jnp.float32),
            pltpu.VMEM((2, _CROWS, _CCOLS), jnp.float32),
            pltpu.VMEM((8, 2), jnp.int32),
            pltpu.SemaphoreType.DMA,
            pltpu.SemaphoreType.DMA,
            pltpu.SemaphoreType.DMA,
        ],
    )(t2, ind, img)


# speedup vs baseline: 1.1829x; 1.1829x over previous
import jax
import jax.numpy as jnp
import numpy as np
from jax.experimental import pallas as pl
from jax.experimental.pallas import tpu as pltpu

_SIZE = 512
_NPOS = 100000
_NROW = 1024
_NLEVELS = 17
_CROWS = 520
_CCOLS = 640

_U = np.uint32(0x3EFA3824).view(np.float32)
_OMU = np.float32(np.float32(1.0) - _U)


def _body(t2_ref, ind_ref, img_ref, out_ref, innert_ref, crop_v,
          ind_v, sem_ind, sem0, sem1):
    acc = jnp.zeros((8, 128), jnp.float32)
    for j in range(128):
        acc = acc + t2_ref[j]
        innert_ref[j] = acc

    si0 = jax.lax.broadcasted_iota(jnp.int32, (8, 128), 0)
    li0 = jax.lax.broadcasted_iota(jnp.int32, (8, 128), 1)
    v = acc
    for j2 in range(1, 128):
        a = j2 % 8
        g = j2 // 8
        moved = pltpu.roll(v, 1, 0)
        if a == 0:
            moved = pltpu.roll(moved, 1, 1)
        v = jnp.where((si0 == a) & (li0 % 16 == g), v + moved, v)
    inner2 = v

    si = jax.lax.broadcasted_iota(jnp.int32, (8, 128), 0)
    li = jax.lax.broadcasted_iota(jnp.int32, (8, 128), 1)

    def _extract(arr, a, b):
        return jnp.sum(jnp.where((si == a) & (li == b), arr,
                                 jnp.zeros_like(arr)))

    bb = inner2
    for k in range(1, 7):
        moved = pltpu.roll(bb, 16, 1)
        bb = jnp.where((si == 7) & (li == 16 * k + 15), bb + moved, bb)
    o2_shift = pltpu.roll(bb, 16, 1)
    y_bc = jnp.where((si == 7) & (li % 16 == 15) & (li >= 16),
                     o2_shift, jnp.float32(0.0))
    for s in (1, 2, 4, 8):
        y_bc = y_bc + pltpu.roll(y_bc, 128 - s, 1)
    for s in (1, 2, 4):
        y_bc = y_bc + pltpu.roll(y_bc, s, 0)

    oiv = y_bc + inner2
    r1s = pltpu.roll(oiv, 1, 0)
    r2s = pltpu.roll(r1s, 1, 1)
    oev = jnp.where(si == 0, r2s, r1s)
    oev = jnp.where((si == 0) & (li == 0), jnp.float32(0.0), oev)

    s_total = _extract(oev + innert_ref[95], 5, 97)
    r_thr = s_total * _OMU

    r_bc = jnp.full((8, 128), r_thr)
    rlin = li * 8 + si
    m_le95 = rlin <= 781
    m_ge96 = rlin <= 780
    big = jnp.full((8, 128), jnp.int32(1 << 30))
    nacc = 4
    cnts = [jnp.zeros((8, 128), jnp.int32) for _ in range(nacc)]
    fges = [big for _ in range(nacc)]
    for j in range(128):
        mj = m_le95 if j <= 95 else m_ge96
        cv = oev + innert_ref[j]
        lt = (cv < r_bc) & mj
        ge = (cv >= r_bc) & mj
        k = j % nacc
        cnts[k] = cnts[k] + jnp.where(lt, 1, 0)
        fges[k] = jnp.minimum(fges[k], jnp.where(ge, rlin * 128 + j, big))
    cnt_all = cnts[0] + cnts[1] + cnts[2] + cnts[3]
    fge_all = jnp.minimum(jnp.minimum(fges[0], fges[1]),
                          jnp.minimum(fges[2], fges[3]))
    a_cnt = jnp.sum(cnt_all)
    first_ge = jnp.min(fge_all)

    def _slow_search():
        def cumval(mid):
            r = mid // 128
            j = mid % 128
            return _extract(oev + innert_ref[j], r % 8, r // 8)
        low = jnp.int32(0)
        high = jnp.int32(_NPOS)
        for _ in range(_NLEVELS):
            mid = low + (high - low) // 2
            go_left = r_thr <= cumval(mid)
            low = jnp.where(go_left, low, mid)
            high = jnp.where(go_left, mid, high)
        return high

    pos = jax.lax.cond(first_ge == a_cnt, lambda: a_cnt, _slow_search)

    pos0 = pl.multiple_of((pos // 8) * 8, 8)
    cp = pltpu.make_async_copy(ind_ref.at[pl.ds(pos0, 8), :], ind_v, sem_ind)
    cp.start()
    cp.wait()
    iv = ind_v[...]
    si2 = jax.lax.broadcasted_iota(jnp.int32, (8, 2), 0)
    li2 = jax.lax.broadcasted_iota(jnp.int32, (8, 2), 1)
    zero = jnp.zeros_like(iv)
    y = jnp.sum(jnp.where((si2 == pos - pos0) & (li2 == 0), iv, zero))
    x = jnp.sum(jnp.where((si2 == pos - pos0) & (li2 == 1), iv, zero))

    y0 = pl.multiple_of((y // 8) * 8, 8)
    x0 = pl.multiple_of((x // 128) * 128, 128)
    dy = y - y0
    dx = x - x0
    c0 = pltpu.make_async_copy(
        img_ref.at[0, pl.ds(y0, _CROWS), pl.ds(x0, _CCOLS)],
        crop_v.at[0], sem0)
    c1 = pltpu.make_async_copy(
        img_ref.at[1, pl.ds(y0, _CROWS), pl.ds(x0, _CCOLS)],
        crop_v.at[1], sem1)
    c0.start()
    c1.start()
    c0.wait()
    c1.wait()
    for c in range(2):
        arr = crop_v[c]
        arr = pltpu.roll(arr, (_CROWS - dy) % _CROWS, 0)
        arr = pltpu.roll(arr, (_CCOLS - dx) % _CCOLS, 1)
        out_ref[c] = arr[:_SIZE, :_SIZE]


def kernel(img, pmap, ind):
    xp = jnp.pad(pmap, (0, _NROW * 128 - _NPOS))
    t2 = xp.reshape(_NROW, 128).T.reshape(128, 128, 8).transpose(0, 2, 1)

    return pl.pallas_call(
        _body,
        in_specs=[
            pl.BlockSpec(memory_space=pltpu.VMEM),
            pl.BlockSpec(memory_space=pl.ANY),
            pl.BlockSpec(memory_space=pl.ANY),
        ],
        out_specs=pl.BlockSpec(memory_space=pltpu.VMEM),
        out_shape=jax.ShapeDtypeStruct((2, _SIZE, _SIZE), jnp.float32),
        scratch_shapes=[
            pltpu.VMEM((128, 8, 128), jnp.float32),
            pltpu.VMEM((2, _CROWS, _CCOLS), jnp.float32),
            pltpu.VMEM((8, 2), jnp.int32),
            pltpu.SemaphoreType.DMA,
            pltpu.SemaphoreType.DMA,
            pltpu.SemaphoreType.DMA,
        ],
    )(t2, ind, img)
